# Initial kernel scaffold; baseline (speedup 1.0000x reference)
import jax
import jax.numpy as jnp
from jax.experimental import pallas as pl


def kernel(class_emb, rel_emb, nf1, nf2, nf3, nf4, disjoint, nf3_neg):
    raise NotImplementedError("write your pallas kernel here")


# speedup vs baseline: 1.3899x; 1.3899x over previous
import functools

import jax
import jax.numpy as jnp
import numpy as np
from jax import lax
from jax.experimental import pallas as pl
from jax.experimental.pallas import tpu as pltpu
from jax.experimental.pallas import tpu_sc as plsc

_BATCH = 4096
_NROWS = 100000
_DIM = 128
_NW = 32
_RPW = _BATCH // _NW
_L = 16
_NSTAT = 16


def _tf2x32(k1, k2, x0, x1):
    rotations = ((13, 15, 26, 6), (17, 29, 16, 24))
    ks = (np.uint32(k1), np.uint32(k2),
          np.uint32(k1) ^ np.uint32(k2) ^ np.uint32(0x1BD11BDA))
    x0 = x0.astype(np.uint32) + ks[0]
    x1 = x1.astype(np.uint32) + ks[1]
    with np.errstate(over="ignore"):
        for d in range(5):
            for r in rotations[d % 2]:
                x0 = x0 + x1
                x1 = (x1 << np.uint32(r)) | (x1 >> np.uint32(32 - r))
                x1 = x1 ^ x0
            x0 = x0 + ks[(d + 1) % 3]
            x1 = x1 + ks[(d + 2) % 3] + np.uint32(d + 1)
    return x0, x1


def _sample_indices_np(seed, batch, maxval):
    f0, f1 = _tf2x32(np.uint32(0), np.uint32(1),
                     np.uint32([0]), np.uint32([seed]))
    s0, s1 = _tf2x32(f0[0], f1[0], np.uint32([0, 0]), np.uint32([0, 1]))
    ar = np.arange(batch, dtype=np.uint32)
    zr = np.zeros(batch, dtype=np.uint32)
    o0, o1 = _tf2x32(s0[0], s1[0], zr, ar)
    y = o0 ^ o1
    o0, o1 = _tf2x32(s0[1], s1[1], zr, ar)
    z = o0 ^ o1
    span = np.uint32(maxval)
    with np.errstate(over="ignore"):
        mult = (np.uint32(65536 % maxval) * np.uint32(65536 % maxval)) % span
        b = ((y % span) * mult + (z % span)) % span
    return b.astype(np.int32)


@functools.lru_cache(maxsize=None)
def _flat_offsets():
    s = [_sample_indices_np(i, _BATCH, _NROWS) for i in range(6)]
    rows = [
        s[0] * 2 + 0, s[0] * 2 + 1,
        s[1] * 3 + 0, s[1] * 3 + 1, s[1] * 3 + 2,
        s[2] * 3 + 0, s[2] * 3 + 1, s[2] * 3 + 2,
        s[3] * 3 + 0, s[3] * 3 + 1, s[3] * 3 + 2,
        s[4] * 2 + 0, s[4] * 2 + 1,
        s[5] * 3 + 0, s[5] * 3 + 1, s[5] * 3 + 2,
    ]
    return np.stack(rows).astype(np.int32)


def _splat(v):
    return lax.broadcast_in_dim(v, (_L,), ())


_DNUMS = lax.GatherDimensionNumbers(offset_dims=(), collapsed_slice_dims=(0,),
                                    start_index_map=(0,))


def _allsum(v):
    for step in (1, 2, 4, 8):
        perm = lax.iota(jnp.int32, _L) ^ step
        g = lax.gather(v, perm[:, None], _DNUMS, slice_sizes=(1,),
                       mode=lax.GatherScatterMode.PROMISE_IN_BOUNDS)
        v = v + g
    return v


_LOSSES = (
    (0, 1, None, "l1"),
    (2, 3, 4, "l2"),
    (5, 7, 6, "relsum"),
    (9, 10, 8, "reldiff"),
    (11, 12, None, "dj"),
    (13, 15, 14, "neg"),
)


def _sc_body(xs_h, rad_h, rel_h, nf1_h, nf2_h, nf3_h, nf4_h, dj_h, neg_h,
             fidx_h, stats_h,
             fv_a, fv_b, fv_c, cid_a, cid_b, cid_c,
             buf_a, buf_b, buf_c, rad_a, rad_b, rad_c,
             stats_v, sem_a, sem_b, sem_c, sem_s):
    cid = lax.axis_index("c")
    sid = lax.axis_index("s")
    wid = sid * 2 + cid
    base = wid * _RPW
    lanes = lax.iota(jnp.int32, _L)
    zero = jnp.zeros((_L,), jnp.float32)
    nf_tabs = (nf1_h, nf2_h, nf3_h, nf4_h, dj_h, neg_h)

    for li, (row_a, row_b, row_t, kind) in enumerate(_LOSSES):
        nf_h = nf_tabs[li]
        third = row_t is not None
        pltpu.sync_copy(fidx_h.at[row_a, pl.ds(base, _RPW)], fv_a)
        pltpu.sync_copy(fidx_h.at[row_b, pl.ds(base, _RPW)], fv_b)
        if third:
            pltpu.sync_copy(fidx_h.at[row_t, pl.ds(base, _RPW)], fv_c)
        d1 = pltpu.async_copy(nf_h.at[fv_a], cid_a, sem_a)
        d2 = pltpu.async_copy(nf_h.at[fv_b], cid_b, sem_b)
        if third:
            d3 = pltpu.async_copy(nf_h.at[fv_c], cid_c, sem_c)
        d1.wait()
        d2.wait()
        if third:
            d3.wait()
        g1 = pltpu.async_copy(xs_h.at[cid_a], buf_a, sem_a)
        g1r = pltpu.async_copy(rad_h.at[cid_a], rad_a.at[pl.ds(0, _RPW)],
                               sem_a)
        g2 = pltpu.async_copy(xs_h.at[cid_b], buf_b, sem_b)
        g2r = pltpu.async_copy(rad_h.at[cid_b], rad_b.at[pl.ds(0, _RPW)],
                               sem_b)
        if third:
            tab3 = xs_h if kind == "l2" else rel_h
            g3 = pltpu.async_copy(tab3.at[cid_c], buf_c, sem_c)
            if kind == "l2":
                g3r = pltpu.async_copy(rad_h.at[cid_c],
                                       rad_c.at[pl.ds(0, _RPW)], sem_c)
        g1.wait()
        g1r.wait()
        g2.wait()
        g2r.wait()
        if third:
            g3.wait()
            if kind == "l2":
                g3r.wait()

        def group(g, carry, kind=kind):
            gb = g * _L

            if kind == "l1":
                def rowfn(j, carry):
                    s1v, sav, sbv = carry
                    i = gb + j
                    rs = jnp.abs(rad_a[pl.ds(i, _L)][0]) - \
                        jnp.abs(rad_b[pl.ds(i, _L)][0])
                    rcd = _splat(rs)
                    s1 = zero
                    sa = zero
                    sb = zero
                    for k in range(8):
                        va = buf_a[i, pl.ds(k * _L, _L)]
                        vb = buf_b[i, pl.ds(k * _L, _L)]
                        s1 = s1 + jnp.maximum(jnp.abs(va - vb) + rcd, 0.0)
                        sa = sa + va * va
                        sb = sb + vb * vb
                    m = lanes == j
                    s1v = jnp.where(m, _allsum(s1), s1v)
                    sav = jnp.where(m, _allsum(sa), sav)
                    sbv = jnp.where(m, _allsum(sb), sbv)
                    return (s1v, sav, sbv)

                s1v, sav, sbv = lax.fori_loop(0, _L, rowfn,
                                              (zero, zero, zero))
                stats_v[0, pl.ds(gb, _L)] = s1v
                stats_v[3, pl.ds(gb, _L)] = sav
                stats_v[4, pl.ds(gb, _L)] = sbv
                return carry

            if kind == "l2":
                def rowfn(j, carry):
                    nab, nac, nbc, sav, sbv, scv = carry
                    i = gb + j
                    ab = zero
                    ac = zero
                    bc = zero
                    sa = zero
                    sb = zero
                    sc = zero
                    for k in range(8):
                        va = buf_a[i, pl.ds(k * _L, _L)]
                        vb = buf_b[i, pl.ds(k * _L, _L)]
                        vc = buf_c[i, pl.ds(k * _L, _L)]
                        dab = vb - va
                        dac = vc - va
                        dbc = vc - vb
                        ab = ab + dab * dab
                        ac = ac + dac * dac
                        bc = bc + dbc * dbc
                        sa = sa + va * va
                        sb = sb + vb * vb
                        sc = sc + vc * vc
                    m = lanes == j
                    nab = jnp.where(m, _allsum(ab), nab)
                    nac = jnp.where(m, _allsum(ac), nac)
                    nbc = jnp.where(m, _allsum(bc), nbc)
                    sav = jnp.where(m, _allsum(sa), sav)
                    sbv = jnp.where(m, _allsum(sb), sbv)
                    scv = jnp.where(m, _allsum(sc), scv)
                    return (nab, nac, nbc, sav, sbv, scv)

                nab, nac, nbc, sav, sbv, scv = lax.fori_loop(
                    0, _L, rowfn, (zero,) * 6)
                stats_v[0, pl.ds(gb, _L)] = nab
                stats_v[1, pl.ds(gb, _L)] = nac
                stats_v[2, pl.ds(gb, _L)] = nbc
                stats_v[3, pl.ds(gb, _L)] = sav
                stats_v[4, pl.ds(gb, _L)] = sbv
                stats_v[5, pl.ds(gb, _L)] = scv
                return carry

            sgn = 1.0 if kind in ("relsum", "neg") else -1.0
            has_rel = kind in ("relsum", "reldiff", "neg")

            def rowfn(j, carry):
                nv, sav, sbv = carry
                i = gb + j
                sn = zero
                sa = zero
                sb = zero
                for k in range(8):
                    va = buf_a[i, pl.ds(k * _L, _L)]
                    vb = buf_b[i, pl.ds(k * _L, _L)]
                    if has_rel:
                        vr = buf_c[i, pl.ds(k * _L, _L)]
                        dn = va + sgn * vr - vb
                    else:
                        dn = vb - va
                    sn = sn + dn * dn
                    sa = sa + va * va
                    sb = sb + vb * vb
                m = lanes == j
                nv = jnp.where(m, _allsum(sn), nv)
                sav = jnp.where(m, _allsum(sa), sav)
                sbv = jnp.where(m, _allsum(sb), sbv)
                return (nv, sav, sbv)

            nv, sav, sbv = lax.fori_loop(0, _L, rowfn, (zero, zero, zero))
            stats_v[0, pl.ds(gb, _L)] = nv
            stats_v[3, pl.ds(gb, _L)] = sav
            stats_v[4, pl.ds(gb, _L)] = sbv
            return carry

        lax.fori_loop(0, _RPW // _L, group, jnp.int32(0))

        used = {"l1": (0, 3, 4), "l2": (0, 1, 2, 3, 4, 5)}.get(kind,
                                                               (0, 3, 4))
        descs = [pltpu.async_copy(stats_v.at[s, pl.ds(0, _RPW)],
                                  stats_h.at[li, s, pl.ds(base, _RPW)],
                                  sem_s) for s in used]
        if kind != "l1":
            descs.append(pltpu.async_copy(rad_a.at[pl.ds(0, _RPW)],
                                          stats_h.at[li, 6,
                                                     pl.ds(base, _RPW)],
                                          sem_s))
            descs.append(pltpu.async_copy(rad_b.at[pl.ds(0, _RPW)],
                                          stats_h.at[li, 7,
                                                     pl.ds(base, _RPW)],
                                          sem_s))
            if kind == "l2":
                descs.append(pltpu.async_copy(rad_c.at[pl.ds(0, _RPW)],
                                              stats_h.at[li, 8,
                                                         pl.ds(base, _RPW)],
                                              sem_s))
        for d in descs:
            d.wait()


def _tc_body(stats_ref, out_ref):
    relu = jax.nn.relu

    def reg(ss):
        return jnp.abs(jnp.sqrt(ss) - 1.0)

    total = jnp.float32(0.0)
    for li, (_, _, _, kind) in enumerate(_LOSSES):
        s0 = stats_ref[li, 0, :]
        sa = stats_ref[li, 3, :]
        sb = stats_ref[li, 4, :]
        if kind == "l1":
            total += jnp.mean(s0) / _DIM + jnp.mean(reg(sa) + reg(sb))
            continue
        ra = jnp.abs(stats_ref[li, 6, :])
        rb = jnp.abs(stats_ref[li, 7, :])
        if kind == "l2":
            nab = jnp.sqrt(s0)
            nac = jnp.sqrt(stats_ref[li, 1, :])
            nbc = jnp.sqrt(stats_ref[li, 2, :])
            rc = jnp.abs(stats_ref[li, 8, :])
            sc = stats_ref[li, 5, :]
            term = (relu(nab - (ra + rb)) + relu(nac - ra)
                    + relu(nbc - rb) + relu(jnp.minimum(ra, rb) - rc)
                    + reg(sa) + reg(sb) + reg(sc))
        else:
            n = jnp.sqrt(s0)
            if kind == "relsum":
                t = relu(n + ra - rb)
            elif kind == "reldiff":
                t = relu(n - ra - rb)
            elif kind == "dj":
                t = relu(ra + rb - n)
            else:
                t = ra + rb - n
            term = t + reg(sa) + reg(sb)
        total += jnp.mean(term)
    out_ref[0, 0] = total


def kernel(class_emb, rel_emb, nf1, nf2, nf3, nf4, disjoint, nf3_neg):
    class_emb = class_emb.astype(jnp.float32)
    xs = class_emb[:, :_DIM]
    rad = class_emb[:, _DIM]
    rel = rel_emb.astype(jnp.float32)
    nfs = [a.astype(jnp.int32).reshape(-1)
           for a in (nf1, nf2, nf3, nf4, disjoint, nf3_neg)]
    fidx = jnp.asarray(_flat_offsets())

    mesh = plsc.VectorSubcoreMesh(
        core_axis_name="c", subcore_axis_name="s", num_cores=2,
        num_subcores=16)
    sc_run = pl.kernel(
        _sc_body,
        out_type=jax.ShapeDtypeStruct((6, _NSTAT, _BATCH), jnp.float32),
        mesh=mesh,
        scratch_types=[
            pltpu.VMEM((_RPW,), jnp.int32),
            pltpu.VMEM((_RPW,), jnp.int32),
            pltpu.VMEM((_RPW,), jnp.int32),
            pltpu.VMEM((_RPW,), jnp.int32),
            pltpu.VMEM((_RPW,), jnp.int32),
            pltpu.VMEM((_RPW,), jnp.int32),
            pltpu.VMEM((_RPW, _DIM), jnp.float32),
            pltpu.VMEM((_RPW, _DIM), jnp.float32),
            pltpu.VMEM((_RPW, _DIM), jnp.float32),
            pltpu.VMEM((_RPW + _L,), jnp.float32),
            pltpu.VMEM((_RPW + _L,), jnp.float32),
            pltpu.VMEM((_RPW + _L,), jnp.float32),
            pltpu.VMEM((_NSTAT, _RPW + _L), jnp.float32),
            pltpu.SemaphoreType.DMA,
            pltpu.SemaphoreType.DMA,
            pltpu.SemaphoreType.DMA,
            pltpu.SemaphoreType.DMA,
        ],
    )
    stats = sc_run(xs, rad, rel, *nfs, fidx)

    total = pl.pallas_call(
        _tc_body,
        out_shape=jax.ShapeDtypeStruct((1, 1), jnp.float32),
        out_specs=pl.BlockSpec(memory_space=pltpu.SMEM),
    )(stats)
    return total[0, 0]
